# baseline (speedup 1.0000x reference)
import functools

import jax
import jax.numpy as jnp
from jax import lax
from jax.experimental import pallas as pl
from jax.experimental.pallas import tpu as pltpu
from jax.experimental.pallas import tpu_sc as plsc

B, F, V, K = 16384, 26, 1000000, 10
NC, NS = 2, 16
NW = NC * NS
SPW = B // NW
CH = 64
NCH = SPW // CH
CF = CH * F
NG = CH // 16
GRAN_W = V // 16

PCH = 2000
NPCH = V // PCH
PPT = (NPCH + NW - 1) // NW


KW = K + 1


def _pack_body(*refs):
    cols = refs[:KW]
    pt_hbm = refs[KW]
    colbuf, rowbuf, sems, osems = refs[KW + 1:]
    c = lax.axis_index("c")
    s = lax.axis_index("s")
    wid = s * NC + c
    iota = lax.iota(jnp.int32, 16)

    def stage(slot, cid):
        @pl.when(cid < NPCH)
        def _():
            base = cid * PCH
            for k in range(KW):
                pltpu.async_copy(cols[k].at[pl.ds(base, PCH)],
                                 colbuf.at[slot, k], sems.at[slot])

    def process(slot, cid, first):
        @pl.when(cid < NPCH)
        def _():
            base = cid * PCH
            for k in range(KW):
                pltpu.make_async_copy(cols[k].at[pl.ds(base, PCH)],
                                      colbuf.at[slot, k],
                                      sems.at[slot]).wait()

            @pl.when(jnp.logical_not(first))
            def _():
                pltpu.make_async_copy(
                    rowbuf.at[slot], pt_hbm.at[pl.ds(0, PCH * 16)],
                    osems.at[slot]).wait()

            def fill(u, _):
                pos = (u * 16 + iota) * 16
                for k in range(KW):
                    xc = colbuf[slot, k, pl.ds(u * 16, 16)]
                    plsc.store_scatter(rowbuf.at[slot], [pos + k], xc)
                return 0

            lax.fori_loop(0, PCH // 16, fill, 0)
            pltpu.async_copy(rowbuf.at[slot],
                             pt_hbm.at[pl.ds(base * 16, PCH * 16)],
                             osems.at[slot])

    stage(0, wid)

    def superstep(t, _):
        cid_a = (2 * t) * NW + wid
        cid_b = (2 * t + 1) * NW + wid
        stage(1, cid_b)
        process(0, cid_a, t == 0)

        @pl.when(t < PPT // 2 - 1)
        def _():
            stage(0, (2 * t + 2) * NW + wid)

        process(1, cid_b, t == 0)
        return 0

    lax.fori_loop(0, PPT // 2, superstep, 0)
    for slot in range(2):
        pltpu.make_async_copy(rowbuf.at[slot],
                              pt_hbm.at[pl.ds(0, PCH * 16)],
                              osems.at[slot]).wait()


_scpack = functools.partial(
    pl.kernel,
    out_type=jax.ShapeDtypeStruct((16 * V,), jnp.float32),
    mesh=plsc.VectorSubcoreMesh(core_axis_name="c", subcore_axis_name="s"),
    scratch_types=[
        pltpu.VMEM((2, KW, PCH), jnp.float32),
        pltpu.VMEM((2, 16 * PCH), jnp.float32),
        pltpu.SemaphoreType.DMA((2,)),
        pltpu.SemaphoreType.DMA((2,)),
    ],
    compiler_params=pltpu.CompilerParams(
        needs_layout_passes=False, use_tc_tiling_on_sc=False),
)(_pack_body)


def _fm_body(idx_hbm, vals_hbm, v_hbm, b_hbm, out_hbm,
             idx_v, vals_v, buf, b_v, out_v, sems):
    c = lax.axis_index("c")
    s = lax.axis_index("s")
    wid = s * NC + c

    pltpu.sync_copy(b_hbm, b_v)
    pltpu.sync_copy(idx_hbm.at[pl.ds(wid * NCH, NCH), :], idx_v)
    pltpu.sync_copy(vals_hbm.at[pl.ds(wid * NCH, NCH), :], vals_v)
    bvec = b_v[...]
    iota = lax.iota(jnp.int32, 16)
    kcols = [jnp.full((16,), k, dtype=jnp.int32) for k in range(K + 1)]

    def stage_in(slot, ch):
        pltpu.async_copy(v_hbm.at[idx_v.at[ch]], buf.at[slot],
                         sems.at[slot])

    def wait_in(slot):
        pltpu.make_async_copy(
            v_hbm.at[idx_v.at[0]], buf.at[slot], sems.at[slot]).wait()

    def compute(slot, ch):
        def group(g, _):
            rbase = (g * 16 + iota) * F

            def fstep(f, carry):
                accw = carry[0]
                acc = list(carry[1:1 + K])
                acc2 = list(carry[1 + K:])
                r = rbase + f
                vf = plsc.load_gather(vals_v.at[ch], [r])
                wv = plsc.load_gather(buf.at[slot], [r, kcols[K]])
                accw = accw + vf * wv
                for k in range(K):
                    x = plsc.load_gather(buf.at[slot], [r, kcols[k]])
                    t = vf * x
                    acc[k] = acc[k] + t
                    acc2[k] = acc2[k] + t * t
                return (accw,) + tuple(acc) + tuple(acc2)

            zf = jnp.zeros((16,), jnp.float32)
            carry = lax.fori_loop(0, F, fstep, (zf,) * (1 + 2 * K))
            accw = carry[0]
            p = zf
            for k in range(K):
                p = p + (carry[1 + k] * carry[1 + k] - carry[1 + K + k])
            logit = accw + bvec + 0.5 * p
            y = 1.0 / (1.0 + jnp.exp(-logit))
            out_v[pl.ds(ch * CH + g * 16, 16)] = y
            return 0

        lax.fori_loop(0, NG, group, 0)

    stage_in(0, 0)

    def superstep(t, _):
        c0 = 2 * t
        stage_in(1, c0 + 1)
        wait_in(0)
        compute(0, c0)

        @pl.when(t < NCH // 2 - 1)
        def _():
            stage_in(0, c0 + 2)

        wait_in(1)
        compute(1, c0 + 1)
        return 0

    lax.fori_loop(0, NCH // 2, superstep, 0)
    pltpu.sync_copy(out_v, out_hbm.at[pl.ds(wid * SPW, SPW)])


@functools.partial(
    pl.kernel,
    out_type=jax.ShapeDtypeStruct((B,), jnp.float32),
    mesh=plsc.VectorSubcoreMesh(core_axis_name="c", subcore_axis_name="s"),
    scratch_types=[
        pltpu.VMEM((NCH, CF), jnp.int32),
        pltpu.VMEM((NCH, CF), jnp.float32),
        pltpu.VMEM((2, CF, 16), jnp.float32),
        pltpu.VMEM((16,), jnp.float32),
        pltpu.VMEM((SPW,), jnp.float32),
        pltpu.SemaphoreType.DMA((2,)),
    ],
    compiler_params=pltpu.CompilerParams(
        needs_layout_passes=False, use_tc_tiling_on_sc=False),
)
def _fm_kernel(idx_hbm, vals_hbm, v_hbm, b_hbm, out_hbm, *rest):
    _fm_body(idx_hbm, vals_hbm, v_hbm, b_hbm, out_hbm, *rest)


def kernel(indices, values, w, v, b):
    idx2 = indices.reshape(B * F // CF, CF).astype(jnp.int32)
    vals2 = values.reshape(B * F // CF, CF).astype(jnp.float32)
    cols = [v[:, k] for k in range(K)] + [w[:, 0]]
    pt = _scpack(*cols).reshape(V, 16)
    b16 = jnp.broadcast_to(b.astype(jnp.float32).reshape(1), (16,))
    return _fm_kernel(idx2, vals2, pt, b16)


# speedup vs baseline: 1.0034x; 1.0034x over previous
import functools

import jax
import jax.numpy as jnp
from jax import lax
from jax.experimental import pallas as pl
from jax.experimental.pallas import tpu as pltpu
from jax.experimental.pallas import tpu_sc as plsc

B, F, V, K = 16384, 26, 1000000, 10
NC, NS = 2, 16
NW = NC * NS
SPW = B // NW
CH = 32
NCH = SPW // CH
CF = CH * F
NG = CH // 16
GRAN_W = V // 16

PCH = 2000
NPCH = V // PCH
PPT = (NPCH + NW - 1) // NW


KW = K + 1


def _pack_body(*refs):
    cols = refs[:KW]
    pt_hbm = refs[KW]
    colbuf, rowbuf, sems, osems = refs[KW + 1:]
    c = lax.axis_index("c")
    s = lax.axis_index("s")
    wid = s * NC + c
    iota = lax.iota(jnp.int32, 16)

    def stage(slot, cid):
        @pl.when(cid < NPCH)
        def _():
            base = cid * PCH
            for k in range(KW):
                pltpu.async_copy(cols[k].at[pl.ds(base, PCH)],
                                 colbuf.at[slot, k], sems.at[slot])

    def process(slot, cid, first):
        @pl.when(cid < NPCH)
        def _():
            base = cid * PCH
            for k in range(KW):
                pltpu.make_async_copy(cols[k].at[pl.ds(base, PCH)],
                                      colbuf.at[slot, k],
                                      sems.at[slot]).wait()

            @pl.when(jnp.logical_not(first))
            def _():
                pltpu.make_async_copy(
                    rowbuf.at[slot], pt_hbm.at[pl.ds(0, PCH * 16)],
                    osems.at[slot]).wait()

            def fill(u, _):
                pos = (u * 16 + iota) * 16
                for k in range(KW):
                    xc = colbuf[slot, k, pl.ds(u * 16, 16)]
                    plsc.store_scatter(rowbuf.at[slot], [pos + k], xc)
                return 0

            lax.fori_loop(0, PCH // 16, fill, 0)
            pltpu.async_copy(rowbuf.at[slot],
                             pt_hbm.at[pl.ds(base * 16, PCH * 16)],
                             osems.at[slot])

    stage(0, wid)

    def superstep(t, _):
        cid_a = (2 * t) * NW + wid
        cid_b = (2 * t + 1) * NW + wid
        stage(1, cid_b)
        process(0, cid_a, t == 0)

        @pl.when(t < PPT // 2 - 1)
        def _():
            stage(0, (2 * t + 2) * NW + wid)

        process(1, cid_b, t == 0)
        return 0

    lax.fori_loop(0, PPT // 2, superstep, 0)
    for slot in range(2):
        pltpu.make_async_copy(rowbuf.at[slot],
                              pt_hbm.at[pl.ds(0, PCH * 16)],
                              osems.at[slot]).wait()


_scpack = functools.partial(
    pl.kernel,
    out_type=jax.ShapeDtypeStruct((16 * V,), jnp.float32),
    mesh=plsc.VectorSubcoreMesh(core_axis_name="c", subcore_axis_name="s"),
    scratch_types=[
        pltpu.VMEM((2, KW, PCH), jnp.float32),
        pltpu.VMEM((2, 16 * PCH), jnp.float32),
        pltpu.SemaphoreType.DMA((2,)),
        pltpu.SemaphoreType.DMA((2,)),
    ],
    compiler_params=pltpu.CompilerParams(
        needs_layout_passes=False, use_tc_tiling_on_sc=False),
)(_pack_body)


def _fm_body(idx_hbm, vals_hbm, v_hbm, b_hbm, out_hbm,
             idx_v, vals_v, buf, b_v, out_v, sems):
    c = lax.axis_index("c")
    s = lax.axis_index("s")
    wid = s * NC + c

    pltpu.sync_copy(b_hbm, b_v)
    pltpu.sync_copy(idx_hbm.at[pl.ds(wid * NCH, NCH), :], idx_v)
    pltpu.sync_copy(vals_hbm.at[pl.ds(wid * NCH, NCH), :], vals_v)
    bvec = b_v[...]
    iota = lax.iota(jnp.int32, 16)
    kcols = [jnp.full((16,), k, dtype=jnp.int32) for k in range(K + 1)]

    def stage_in(slot, ch):
        pltpu.async_copy(v_hbm.at[idx_v.at[ch]], buf.at[slot],
                         sems.at[slot])

    def wait_in(slot):
        pltpu.make_async_copy(
            v_hbm.at[idx_v.at[0]], buf.at[slot], sems.at[slot]).wait()

    def compute(slot, ch):
        def group(g, _):
            rbase = (g * 16 + iota) * F

            def fstep(f, carry):
                accw = carry[0]
                acc = list(carry[1:1 + K])
                acc2 = list(carry[1 + K:])
                r = rbase + f
                vf = plsc.load_gather(vals_v.at[ch], [r])
                wv = plsc.load_gather(buf.at[slot], [r, kcols[K]])
                accw = accw + vf * wv
                for k in range(K):
                    x = plsc.load_gather(buf.at[slot], [r, kcols[k]])
                    t = vf * x
                    acc[k] = acc[k] + t
                    acc2[k] = acc2[k] + t * t
                return (accw,) + tuple(acc) + tuple(acc2)

            zf = jnp.zeros((16,), jnp.float32)
            carry = lax.fori_loop(0, F, fstep, (zf,) * (1 + 2 * K))
            accw = carry[0]
            p = zf
            for k in range(K):
                p = p + (carry[1 + k] * carry[1 + k] - carry[1 + K + k])
            logit = accw + bvec + 0.5 * p
            y = 1.0 / (1.0 + jnp.exp(-logit))
            out_v[pl.ds(ch * CH + g * 16, 16)] = y
            return 0

        lax.fori_loop(0, NG, group, 0)

    stage_in(0, 0)

    def superstep(t, _):
        c0 = 2 * t
        stage_in(1, c0 + 1)
        wait_in(0)
        compute(0, c0)

        @pl.when(t < NCH // 2 - 1)
        def _():
            stage_in(0, c0 + 2)

        wait_in(1)
        compute(1, c0 + 1)
        return 0

    lax.fori_loop(0, NCH // 2, superstep, 0)
    pltpu.sync_copy(out_v, out_hbm.at[pl.ds(wid * SPW, SPW)])


@functools.partial(
    pl.kernel,
    out_type=jax.ShapeDtypeStruct((B,), jnp.float32),
    mesh=plsc.VectorSubcoreMesh(core_axis_name="c", subcore_axis_name="s"),
    scratch_types=[
        pltpu.VMEM((NCH, CF), jnp.int32),
        pltpu.VMEM((NCH, CF), jnp.float32),
        pltpu.VMEM((2, CF, 16), jnp.float32),
        pltpu.VMEM((16,), jnp.float32),
        pltpu.VMEM((SPW,), jnp.float32),
        pltpu.SemaphoreType.DMA((2,)),
    ],
    compiler_params=pltpu.CompilerParams(
        needs_layout_passes=False, use_tc_tiling_on_sc=False),
)
def _fm_kernel(idx_hbm, vals_hbm, v_hbm, b_hbm, out_hbm, *rest):
    _fm_body(idx_hbm, vals_hbm, v_hbm, b_hbm, out_hbm, *rest)


def kernel(indices, values, w, v, b):
    idx2 = indices.reshape(B * F // CF, CF).astype(jnp.int32)
    vals2 = values.reshape(B * F // CF, CF).astype(jnp.float32)
    cols = [v[:, k] for k in range(K)] + [w[:, 0]]
    pt = _scpack(*cols).reshape(V, 16)
    b16 = jnp.broadcast_to(b.astype(jnp.float32).reshape(1), (16,))
    return _fm_kernel(idx2, vals2, pt, b16)
